# Initial kernel scaffold; baseline (speedup 1.0000x reference)
import jax
import jax.numpy as jnp
from jax.experimental import pallas as pl


def kernel(pred_PM, pred_Ms):
    raise NotImplementedError("write your pallas kernel here")


# speedup vs baseline: 106.9441x; 106.9441x over previous
import numpy as np
import jax
import jax.numpy as jnp
from jax.experimental import pallas as pl
from jax.experimental.pallas import tpu as pltpu

_B, _H, _W = 64, 512, 512
_W0 = float(1.0 / np.sqrt(2.0 * np.pi) + 1.0)
_TOTAL = float(_B * _H * _W)


def _loss_kernel(pm_ref, ms_ref, out_ref, acc_ref):
    i = pl.program_id(0)

    @pl.when(i == 0)
    def _():
        acc_ref[0] = 0.0
        acc_ref[1] = 0.0
        acc_ref[2] = 0.0

    pm = pm_ref[...]
    ms = ms_ref[...]
    pos = ms > 0.0
    q = jnp.clip(jnp.where(pos, pm, 1.0 - pm), 1e-7, 1.0 - 1e-7)
    l = -jnp.log(q)
    acc_ref[0] += jnp.sum(l * ms)
    acc_ref[1] += jnp.sum(l * (1.0 - ms))
    acc_ref[2] += jnp.sum(ms)

    @pl.when(i == pl.num_programs(0) - 1)
    def _():
        s_pos = acc_ref[0]
        s_neg = acc_ref[1]
        n_pos = acc_ref[2]
        n_neg = _TOTAL - n_pos
        loss = jnp.where(n_pos > 0.0, s_pos / n_pos, 0.0)
        loss = loss + jnp.where(n_neg > 0.0, s_neg / n_neg, 0.0)
        out_ref[0, 0] = loss * _W0


def kernel(pred_PM, pred_Ms):
    out = pl.pallas_call(
        _loss_kernel,
        grid=(_B,),
        in_specs=[
            pl.BlockSpec((1, _H, _W), lambda i: (i, 0, 0)),
            pl.BlockSpec((1, _H, _W), lambda i: (i, 0, 0)),
        ],
        out_specs=pl.BlockSpec(memory_space=pltpu.SMEM),
        out_shape=jax.ShapeDtypeStruct((1, 1), jnp.float32),
        scratch_shapes=[pltpu.SMEM((3,), jnp.float32)],
    )(pred_PM, pred_Ms)
    return (jnp.zeros((), jnp.float32), out[0, 0])


# speedup vs baseline: 148.5214x; 1.3888x over previous
import numpy as np
import jax
import jax.numpy as jnp
from jax.experimental import pallas as pl
from jax.experimental.pallas import tpu as pltpu

_B, _H, _W = 64, 512, 512
_W0 = float(1.0 / np.sqrt(2.0 * np.pi) + 1.0)
_TOTAL = float(_B * _H * _W)


def _loss_kernel(pm_ref, ms_ref, out_ref, acc_ref):
    i = pl.program_id(0)

    @pl.when(i == 0)
    def _():
        acc_ref[0] = 0.0
        acc_ref[1] = 0.0
        acc_ref[2] = 0.0

    pm = pm_ref[...]
    ms = ms_ref[...]
    pos = ms > 0.0
    q = jnp.clip(jnp.where(pos, pm, 1.0 - pm), 1e-7, 1.0 - 1e-7)
    l = -jnp.log(q)
    acc_ref[0] += jnp.sum(l * ms)
    acc_ref[1] += jnp.sum(l * (1.0 - ms))
    acc_ref[2] += jnp.sum(ms)

    @pl.when(i == pl.num_programs(0) - 1)
    def _():
        s_pos = acc_ref[0]
        s_neg = acc_ref[1]
        n_pos = acc_ref[2]
        n_neg = _TOTAL - n_pos
        loss = jnp.where(n_pos > 0.0, s_pos / n_pos, 0.0)
        loss = loss + jnp.where(n_neg > 0.0, s_neg / n_neg, 0.0)
        out_ref[0, 0] = loss * _W0


def kernel(pred_PM, pred_Ms):
    out = pl.pallas_call(
        _loss_kernel,
        grid=(_B // 4,),
        in_specs=[
            pl.BlockSpec((4, _H, _W), lambda i: (i, 0, 0)),
            pl.BlockSpec((4, _H, _W), lambda i: (i, 0, 0)),
        ],
        out_specs=pl.BlockSpec(memory_space=pltpu.SMEM),
        out_shape=jax.ShapeDtypeStruct((1, 1), jnp.float32),
        scratch_shapes=[pltpu.SMEM((3,), jnp.float32)],
    )(pred_PM, pred_Ms)
    return (jnp.zeros((), jnp.float32), out[0, 0])
